# Initial kernel scaffold; baseline (speedup 1.0000x reference)
import functools

import jax
import jax.numpy as jnp
from jax import lax
from jax.experimental import pallas as pl
from jax.experimental.pallas import tpu as pltpu
from jax.experimental.pallas import tpu_sc as plsc

N = 10000
E = 320000
D = 128
NP = 10240
NC = 2
NS = 16
EA = E // (NC * NS)
EB = E // NS
CH = 80
ROWB = 1000
EBLK = 2000


def _pad8(col0, col1):
    zeros = jnp.zeros((D, 6), jnp.float32)
    return jnp.concatenate([col0, col1, zeros], axis=1)


def _tc_pre_body(ea_ref, fr0_ref, fr1_ref, a0_ref, a1_ref, s3_ref):
    r0 = jnp.dot(fr0_ref[...], a0_ref[2 * D:3 * D, :],
                 preferred_element_type=jnp.float32)
    r1 = jnp.dot(fr1_ref[...], a1_ref[2 * D:3 * D, :],
                 preferred_element_type=jnp.float32)
    R = _pad8(r0, r1)
    s3_ref[...] = lax.dot_general(R, ea_ref[...], (((0,), (1,)), ((), ())),
                                  preferred_element_type=jnp.float32)


def _tc_pre(edge_attr, fc_r_w0, fc_r_w1, attn_w0, attn_w1):
    wfull = pl.BlockSpec((D, D), lambda b: (0, 0))
    afull = pl.BlockSpec((3 * D, 1), lambda b: (0, 0))
    return pl.pallas_call(
        _tc_pre_body,
        grid=(E // EBLK,),
        in_specs=[pl.BlockSpec((EBLK, D), lambda b: (b, 0)),
                  wfull, wfull, afull, afull],
        out_specs=pl.BlockSpec((8, EBLK), lambda b: (0, b)),
        out_shape=jax.ShapeDtypeStruct((8, E), jnp.float32),
    )(edge_attr, fc_r_w0, fc_r_w1, attn_w0, attn_w1)


def _project(z, loopw_ref, aw_ref, z_ref, zl_ref, s12_ref):
    z_ref[...] = z
    zl_ref[...] = jnp.dot(z, loopw_ref[...], preferred_element_type=jnp.float32)
    W = _pad8(aw_ref[0:D, :], aw_ref[D:2 * D, :])
    s12_ref[...] = lax.dot_general(W, z, (((0,), (1,)), ((), ())),
                                   preferred_element_type=jnp.float32)


def _tc0_body(x_ref, fcw_ref, loopw_ref, aw_ref, z_ref, zl_ref, s12_ref):
    z = jnp.dot(x_ref[...], fcw_ref[...], preferred_element_type=jnp.float32)
    _project(z, loopw_ref, aw_ref, z_ref, zl_ref, s12_ref)


def _tc1_body(a1_ref, a2_ref, frw_ref, zlp_ref, fcw_ref, loopw_ref, aw_ref,
              z_ref, zl_ref, s12_ref):
    h = jnp.maximum(
        a1_ref[...]
        + jnp.dot(a2_ref[...], frw_ref[...], preferred_element_type=jnp.float32)
        + zlp_ref[...], 0.0)
    z = jnp.dot(h, fcw_ref[...], preferred_element_type=jnp.float32)
    _project(z, loopw_ref, aw_ref, z_ref, zl_ref, s12_ref)


def _tc2_body(a1_ref, a2_ref, frw_ref, zlp_ref, out_ref):
    out_ref[...] = jnp.maximum(
        a1_ref[...]
        + jnp.dot(a2_ref[...], frw_ref[...], preferred_element_type=jnp.float32)
        + zlp_ref[...], 0.0)


_ROWBS = pl.BlockSpec((ROWB, D), lambda b: (b, 0))
_WBS = pl.BlockSpec((D, D), lambda b: (0, 0))
_ABS = pl.BlockSpec((3 * D, 1), lambda b: (0, 0))
_PROJ_OUT = dict(
    out_specs=[pl.BlockSpec((ROWB, D), lambda b: (b, 0)),
               pl.BlockSpec((ROWB, D), lambda b: (b, 0)),
               pl.BlockSpec((8, ROWB), lambda b: (0, b))],
    out_shape=[jax.ShapeDtypeStruct((N, D), jnp.float32),
               jax.ShapeDtypeStruct((N, D), jnp.float32),
               jax.ShapeDtypeStruct((8, N), jnp.float32)],
)


def _tc0(x, fc_w, loop_w, attn_w):
    return pl.pallas_call(
        _tc0_body, grid=(N // ROWB,),
        in_specs=[_ROWBS, _WBS, _WBS, _ABS], **_PROJ_OUT,
    )(x, fc_w, loop_w, attn_w)


def _tc1(a1, a2, fc_r_w, zl_prev, fc_w, loop_w, attn_w):
    return pl.pallas_call(
        _tc1_body, grid=(N // ROWB,),
        in_specs=[_ROWBS, _ROWBS, _WBS, _ROWBS, _WBS, _WBS, _ABS], **_PROJ_OUT,
    )(a1, a2, fc_r_w, zl_prev, fc_w, loop_w, attn_w)


def _tc2(a1, a2, fc_r_w, zl_prev):
    return pl.pallas_call(
        _tc2_body, grid=(N // ROWB,),
        in_specs=[_ROWBS, _ROWBS, _WBS, _ROWBS],
        out_specs=pl.BlockSpec((ROWB, D), lambda b: (b, 0)),
        out_shape=jax.ShapeDtypeStruct((N, D), jnp.float32),
    )(a1, a2, fc_r_w, zl_prev)


_MESH = plsc.VectorSubcoreMesh(core_axis_name="c", subcore_axis_name="s",
                               num_cores=NC, num_subcores=NS)


@functools.partial(
    pl.kernel,
    out_type=[jax.ShapeDtypeStruct((E,), jnp.float32),
              jax.ShapeDtypeStruct((NC, NP), jnp.float32)],
    mesh=_MESH,
    scratch_types=[
        pltpu.VMEM((N,), jnp.float32),
        pltpu.VMEM((N,), jnp.float32),
        pltpu.VMEM((EA,), jnp.int32),
        pltpu.VMEM((EA,), jnp.int32),
        pltpu.VMEM((EA,), jnp.float32),
        pltpu.VMEM((EA,), jnp.float32),
        pltpu.VMEM((CH,), jnp.int32),
        pltpu.VMEM((640,), jnp.float32),
        pltpu.VMEM_SHARED((NP,), jnp.float32),
    ],
)
def _phase_a(src_h, dst_h, s3_h, s1_h, s2_h, ex_h, den_h,
             s1v, s2v, srcv, dstv, s3v, exv, idxv, zerov, dshared):
    c = lax.axis_index("c")
    s = lax.axis_index("s")
    wid = c * NS + s
    base = wid * EA

    pltpu.sync_copy(s1_h, s1v)
    pltpu.sync_copy(s2_h, s2v)
    pltpu.sync_copy(src_h.at[pl.ds(base, EA)], srcv)
    pltpu.sync_copy(dst_h.at[pl.ds(base, EA)], dstv)
    pltpu.sync_copy(s3_h.at[pl.ds(base, EA)], s3v)

    @pl.loop(0, 640 // 16)
    def _zero(i):
        zerov[pl.ds(i * 16, 16)] = jnp.zeros((16,), jnp.float32)

    pltpu.sync_copy(zerov, dshared.at[pl.ds(s * 640, 640)])
    plsc.subcore_barrier()

    @pl.loop(0, EA // 16)
    def _logits(i):
        off = i * 16
        sv = srcv[pl.ds(off, 16)]
        dv = dstv[pl.ds(off, 16)]
        g1 = plsc.load_gather(s1v, [sv])
        g2 = plsc.load_gather(s2v, [dv])
        a = g1 + g2 + s3v[pl.ds(off, 16)]
        e = jnp.maximum(a, 0.01 * a)
        cc = jnp.maximum(g2, 0.01 * g2)
        exv[pl.ds(off, 16)] = jnp.exp(e - cc)

    @pl.loop(0, EA // CH)
    def _denom(j):
        off = j * CH
        for k in range(CH // 16):
            idxv[pl.ds(k * 16, 16)] = dstv[pl.ds(off + k * 16, 16)]
        pltpu.sync_copy(exv.at[pl.ds(off, CH)], dshared.at[idxv], add=True)

    plsc.subcore_barrier()
    pltpu.sync_copy(dshared.at[pl.ds(s * 640, 640)],
                    den_h.at[c, pl.ds(s * 640, 640)])
    pltpu.sync_copy(exv, ex_h.at[pl.ds(base, EA)])


@functools.partial(
    pl.kernel,
    out_type=[jax.ShapeDtypeStruct((NP, D), jnp.float32),
              jax.ShapeDtypeStruct((NP, D), jnp.float32)],
    mesh=_MESH,
    scratch_types=[
        pltpu.VMEM((NP,), jnp.float32),
        pltpu.VMEM((NP,), jnp.float32),
        pltpu.VMEM((EB,), jnp.int32),
        pltpu.VMEM((EB,), jnp.int32),
        pltpu.VMEM((EB,), jnp.float32),
        pltpu.VMEM((CH, D), jnp.float32),
        pltpu.VMEM((64, D), jnp.float32),
        pltpu.VMEM((CH,), jnp.float32),
        pltpu.VMEM((CH,), jnp.int32),
        pltpu.VMEM_SHARED((NP, D), jnp.float32),
    ],
)
def _phase_b(src_h, dst_h, ex_h, den_h, z_h, ea_h, a1_h, a2_h,
             denv, tmpv, srcv, dstv, exv, rowv, zrows, alphav, idxv, aggsh):
    c = lax.axis_index("c")
    s = lax.axis_index("s")
    base = s * EB

    pltpu.sync_copy(den_h.at[0], denv)
    pltpu.sync_copy(den_h.at[1], tmpv)

    @pl.loop(0, NP // 16)
    def _sumden(i):
        o = i * 16
        denv[pl.ds(o, 16)] = denv[pl.ds(o, 16)] + tmpv[pl.ds(o, 16)]

    pltpu.sync_copy(dst_h.at[pl.ds(base, EB)], dstv)
    pltpu.sync_copy(ex_h.at[pl.ds(base, EB)], exv)

    @pl.when(c == 0)
    def _():
        pltpu.sync_copy(src_h.at[pl.ds(base, EB)], srcv)

    @pl.loop(0, 64)
    def _zrow(r):
        for q in range(D // 16):
            zrows[r, pl.ds(q * 16, 16)] = jnp.zeros((16,), jnp.float32)

    for k in range(10):
        pltpu.sync_copy(zrows, aggsh.at[pl.ds(s * 640 + k * 64, 64)])
    plsc.subcore_barrier()

    @pl.loop(0, EB // CH)
    def _edges(j):
        off = j * CH
        for k in range(CH // 16):
            o = off + k * 16
            dv = dstv[pl.ds(o, 16)]
            dd = plsc.load_gather(denv, [dv])
            alphav[pl.ds(k * 16, 16)] = exv[pl.ds(o, 16)] / dd

        @pl.when(c == 0)
        def _():
            for k in range(CH // 16):
                idxv[pl.ds(k * 16, 16)] = srcv[pl.ds(off + k * 16, 16)]
            pltpu.sync_copy(z_h.at[idxv], rowv)

        @pl.when(c == 1)
        def _():
            pltpu.sync_copy(ea_h.at[pl.ds(base + off, CH)], rowv)

        @pl.loop(0, CH, unroll=8)
        def _scale(e):
            a = alphav[e]
            for q in range(D // 16):
                rowv[e, pl.ds(q * 16, 16)] = rowv[e, pl.ds(q * 16, 16)] * a

        for k in range(CH // 16):
            idxv[pl.ds(k * 16, 16)] = dstv[pl.ds(off + k * 16, 16)]
        pltpu.sync_copy(rowv, aggsh.at[idxv], add=True)

    plsc.subcore_barrier()

    @pl.when(c == 0)
    def _():
        pltpu.sync_copy(aggsh.at[pl.ds(s * 640, 640)],
                        a1_h.at[pl.ds(s * 640, 640)])

    @pl.when(c == 1)
    def _():
        pltpu.sync_copy(aggsh.at[pl.ds(s * 640, 640)],
                        a2_h.at[pl.ds(s * 640, 640)])


def kernel(x, edge_index, edge_attr, fc_w0, fc_r_w0, attn_w0, loop_w0,
           fc_w1, fc_r_w1, attn_w1, loop_w1):
    src = edge_index[0]
    dst = edge_index[1]

    s3T = _tc_pre(edge_attr, fc_r_w0, fc_r_w1, attn_w0, attn_w1)

    z0, zl0, s12 = _tc0(x, fc_w0, loop_w0, attn_w0)
    ex0, den0 = _phase_a(src, dst, s3T[0], s12[0], s12[1])
    a1_0, a2_0 = _phase_b(src, dst, ex0, den0, z0, edge_attr)

    z1, zl1, s12b = _tc1(a1_0[:N], a2_0[:N], fc_r_w0, zl0,
                         fc_w1, loop_w1, attn_w1)
    ex1, den1 = _phase_a(src, dst, s3T[1], s12b[0], s12b[1])
    a1_1, a2_1 = _phase_b(src, dst, ex1, den1, z1, edge_attr)

    return _tc2(a1_1[:N], a2_1[:N], fc_r_w1, zl1)


# speedup vs baseline: 11.6293x; 11.6293x over previous
import functools

import jax
import jax.numpy as jnp
from jax import lax
from jax.experimental import pallas as pl
from jax.experimental.pallas import tpu as pltpu
from jax.experimental.pallas import tpu_sc as plsc

N = 10000
E = 320000
D = 128
NP = 10240
NC = 2
NS = 16
EB = E // NS
SCHK = 4000
CH = 80
ROWB = 1024
EBLK = 2560


def _pad8(col0, col1):
    zeros = jnp.zeros((D, 6), jnp.float32)
    return jnp.concatenate([col0, col1, zeros], axis=1)


def _tc_pre_body(ea_ref, fr0_ref, fr1_ref, a0_ref, a1_ref, s3_ref):
    r0 = jnp.dot(fr0_ref[...], a0_ref[2 * D:3 * D, :],
                 preferred_element_type=jnp.float32)
    r1 = jnp.dot(fr1_ref[...], a1_ref[2 * D:3 * D, :],
                 preferred_element_type=jnp.float32)
    R = _pad8(r0, r1)
    s3_ref[...] = lax.dot_general(R, ea_ref[...], (((0,), (1,)), ((), ())),
                                  preferred_element_type=jnp.float32)


def _tc_pre(edge_attr, fc_r_w0, fc_r_w1, attn_w0, attn_w1):
    wfull = pl.BlockSpec((D, D), lambda b: (0, 0))
    afull = pl.BlockSpec((3 * D, 1), lambda b: (0, 0))
    return pl.pallas_call(
        _tc_pre_body,
        grid=(E // EBLK,),
        in_specs=[pl.BlockSpec((EBLK, D), lambda b: (b, 0)),
                  wfull, wfull, afull, afull],
        out_specs=pl.BlockSpec((8, EBLK), lambda b: (0, b)),
        out_shape=jax.ShapeDtypeStruct((8, E), jnp.float32),
    )(edge_attr, fc_r_w0, fc_r_w1, attn_w0, attn_w1)


def _project(z, loopw_ref, aw_ref, z_ref, zl_ref, s12_ref):
    z_ref[...] = z
    zl_ref[...] = jnp.dot(z, loopw_ref[...], preferred_element_type=jnp.float32)
    W = _pad8(aw_ref[0:D, :], aw_ref[D:2 * D, :])
    s12_ref[...] = lax.dot_general(W, z, (((0,), (1,)), ((), ())),
                                   preferred_element_type=jnp.float32)


def _combine(a1_ref, a2_ref, den_ref, frw_ref, zlp_ref):
    den = den_ref[...]
    dens = jnp.where(den > 0.0, den, 1.0)
    msg = (a1_ref[...]
           + jnp.dot(a2_ref[...], frw_ref[...],
                     preferred_element_type=jnp.float32)) / dens
    return jnp.maximum(msg + zlp_ref[...], 0.0)


def _tc0_body(x_ref, fcw_ref, loopw_ref, aw_ref, z_ref, zl_ref, s12_ref):
    z = jnp.dot(x_ref[...], fcw_ref[...], preferred_element_type=jnp.float32)
    _project(z, loopw_ref, aw_ref, z_ref, zl_ref, s12_ref)


def _tc1_body(a1_ref, a2_ref, den_ref, frw_ref, zlp_ref, fcw_ref, loopw_ref,
              aw_ref, z_ref, zl_ref, s12_ref):
    h = _combine(a1_ref, a2_ref, den_ref, frw_ref, zlp_ref)
    z = jnp.dot(h, fcw_ref[...], preferred_element_type=jnp.float32)
    _project(z, loopw_ref, aw_ref, z_ref, zl_ref, s12_ref)


def _tc2_body(a1_ref, a2_ref, den_ref, frw_ref, zlp_ref, out_ref):
    out_ref[...] = _combine(a1_ref, a2_ref, den_ref, frw_ref, zlp_ref)


_ROWBS = pl.BlockSpec((ROWB, D), lambda b: (b, 0))
_DENBS = pl.BlockSpec((ROWB, 1), lambda b: (b, 0))
_WBS = pl.BlockSpec((D, D), lambda b: (0, 0))
_ABS = pl.BlockSpec((3 * D, 1), lambda b: (0, 0))
_PROJ_OUT = dict(
    out_specs=[pl.BlockSpec((ROWB, D), lambda b: (b, 0)),
               pl.BlockSpec((ROWB, D), lambda b: (b, 0)),
               pl.BlockSpec((8, ROWB), lambda b: (0, b))],
    out_shape=[jax.ShapeDtypeStruct((NP, D), jnp.float32),
               jax.ShapeDtypeStruct((NP, D), jnp.float32),
               jax.ShapeDtypeStruct((8, NP), jnp.float32)],
)


def _tc0(x, fc_w, loop_w, attn_w):
    return pl.pallas_call(
        _tc0_body, grid=(NP // ROWB,),
        in_specs=[_ROWBS, _WBS, _WBS, _ABS], **_PROJ_OUT,
    )(x, fc_w, loop_w, attn_w)


def _tc1(a1, a2, den, fc_r_w, zl_prev, fc_w, loop_w, attn_w):
    return pl.pallas_call(
        _tc1_body, grid=(NP // ROWB,),
        in_specs=[_ROWBS, _ROWBS, _DENBS, _WBS, _ROWBS, _WBS, _WBS, _ABS],
        **_PROJ_OUT,
    )(a1, a2, den, fc_r_w, zl_prev, fc_w, loop_w, attn_w)


def _tc2(a1, a2, den, fc_r_w, zl_prev):
    return pl.pallas_call(
        _tc2_body, grid=(NP // ROWB,),
        in_specs=[_ROWBS, _ROWBS, _DENBS, _WBS, _ROWBS],
        out_specs=pl.BlockSpec((ROWB, D), lambda b: (b, 0)),
        out_shape=jax.ShapeDtypeStruct((NP, D), jnp.float32),
    )(a1, a2, den, fc_r_w, zl_prev)


_MESH = plsc.VectorSubcoreMesh(core_axis_name="c", subcore_axis_name="s",
                               num_cores=NC, num_subcores=NS)


@functools.partial(
    pl.kernel,
    out_type=jax.ShapeDtypeStruct((2 * NP + 80, D), jnp.float32),
    mesh=_MESH,
    compiler_params=pltpu.CompilerParams(needs_layout_passes=False),
    scratch_types=[
        pltpu.VMEM((NP,), jnp.float32),
        pltpu.VMEM((NP,), jnp.float32),
        pltpu.VMEM((SCHK,), jnp.int32),
        pltpu.VMEM((SCHK,), jnp.int32),
        pltpu.VMEM((SCHK,), jnp.float32),
        pltpu.VMEM((CH, D), jnp.float32),
        pltpu.VMEM((CH,), jnp.int32),
        pltpu.VMEM((1024,), jnp.float32),
        pltpu.VMEM((8, D), jnp.float32),
        pltpu.VMEM_SHARED((NP, D), jnp.float32),
        pltpu.VMEM_SHARED((NP,), jnp.float32),
    ],
)
def _sc_layer(src_h, dst_h, s3_h, s1_h, s2_h, z_h, ea_h, out_h,
              s1v, s2v, srcv, dstv, exv, zbuf, idxv, dtmp, dtmp2,
              aggsh, dshared):
    c = lax.axis_index("c")
    s = lax.axis_index("s")
    base = s * EB

    pltpu.sync_copy(s1_h, s1v)
    pltpu.sync_copy(s2_h, s2v)

    @pl.loop(0, CH)
    def _zrow(r):
        for q in range(D // 16):
            zbuf[r, pl.ds(q * 16, 16)] = jnp.zeros((16,), jnp.float32)

    @pl.loop(0, 1024 // 16)
    def _zden(i):
        dtmp[pl.ds(i * 16, 16)] = jnp.zeros((16,), jnp.float32)

    for k in range(640 // CH):
        pltpu.sync_copy(zbuf, aggsh.at[pl.ds(s * 640 + k * CH, CH)])

    @pl.when(c == 0)
    def _():
        pltpu.sync_copy(dtmp.at[pl.ds(0, 640)], dshared.at[pl.ds(s * 640, 640)])

    plsc.subcore_barrier()

    @pl.loop(0, EB // SCHK)
    def _super(u):
        sbase = base + u * SCHK
        pltpu.sync_copy(dst_h.at[pl.ds(sbase, SCHK)], dstv)
        pltpu.sync_copy(s3_h.at[pl.ds(sbase, SCHK)], exv)
        pltpu.sync_copy(src_h.at[pl.ds(sbase, SCHK)], srcv)

        @pl.loop(0, SCHK // 16)
        def _logits(i):
            off = i * 16
            sv = srcv[pl.ds(off, 16)]
            dv = dstv[pl.ds(off, 16)]
            g2 = plsc.load_gather(s2v, [dv])
            a = plsc.load_gather(s1v, [sv]) + g2 + exv[pl.ds(off, 16)]
            e = jnp.maximum(a, 0.01 * a)
            cc = jnp.maximum(g2, 0.01 * g2)
            exv[pl.ds(off, 16)] = jnp.exp(e - cc)

        @pl.loop(0, SCHK // CH)
        def _edges(j):
            off = j * CH

            @pl.when(c == 0)
            def _():
                for k in range(CH // 16):
                    idxv[pl.ds(k * 16, 16)] = srcv[pl.ds(off + k * 16, 16)]
                pltpu.sync_copy(z_h.at[idxv], zbuf)

            @pl.when(c == 1)
            def _():
                pltpu.sync_copy(ea_h.at[pl.ds(sbase + off, CH)], zbuf)

            @pl.loop(0, CH // 16)
            def _scale(k):
                exl = exv[pl.ds(off + k * 16, 16)]
                for i in range(16):
                    w = exl[i]
                    e = k * 16 + i
                    for q in range(D // 16):
                        zbuf[e, pl.ds(q * 16, 16)] = (
                            zbuf[e, pl.ds(q * 16, 16)] * w)

            for k in range(CH // 16):
                idxv[pl.ds(k * 16, 16)] = dstv[pl.ds(off + k * 16, 16)]
            pltpu.sync_copy(zbuf, aggsh.at[idxv], add=True)

            @pl.when(c == 0)
            def _():
                pltpu.sync_copy(exv.at[pl.ds(off, CH)], dshared.at[idxv],
                                add=True)

    plsc.subcore_barrier()

    @pl.when(c == 0)
    def _():
        pltpu.sync_copy(aggsh.at[pl.ds(s * 640, 640)],
                        out_h.at[pl.ds(s * 640, 640)])

    @pl.when(c == 1)
    def _():
        pltpu.sync_copy(aggsh.at[pl.ds(s * 640, 640)],
                        out_h.at[pl.ds(NP + s * 640, 640)])

    @pl.when(jnp.logical_and(c == 0, s < 10))
    def _():
        pltpu.sync_copy(dshared.at[pl.ds(s * 1024, 1024)], dtmp)

        @pl.loop(0, 8)
        def _d2(r):
            for q in range(D // 16):
                dtmp2[r, pl.ds(q * 16, 16)] = dtmp[pl.ds(r * 128 + q * 16, 16)]

        pltpu.sync_copy(dtmp2, out_h.at[pl.ds(2 * NP + s * 8, 8)])


_USE_SC = dict(a1=True, a2=True, den=True)


def _xla_layer(src, dst, s3row, s1, s2, z, ea):
    a = s1[src] + s2[dst] + s3row
    e = jnp.maximum(a, 0.01 * a)
    cc = jnp.maximum(s2[dst], 0.01 * s2[dst])
    ex = jnp.exp(e - cc)
    den = jax.ops.segment_sum(ex, dst, num_segments=NP)
    agg1 = jax.ops.segment_sum(ex[:, None] * jnp.take(z, src, axis=0), dst,
                               num_segments=NP)
    agg2 = jax.ops.segment_sum(ex[:, None] * ea, dst, num_segments=NP)
    return agg1, agg2, den


def _layer_parts(src, dst, s3row, s1, s2, z, ea):
    agg = _sc_layer(src, dst, s3row, s1, s2, z, ea)
    a1 = agg[:NP]
    a2 = agg[NP:2 * NP]
    den = jnp.reshape(agg[2 * NP:], (NP, 1))
    if not all(_USE_SC.values()):
        xa1, xa2, xden = _xla_layer(src, dst, s3row, s1, s2, z, ea)
        if not _USE_SC["a1"]:
            a1 = xa1
        if not _USE_SC["a2"]:
            a2 = xa2
        if not _USE_SC["den"]:
            den = xden[:, None]
    return a1, a2, den


def kernel(x, edge_index, edge_attr, fc_w0, fc_r_w0, attn_w0, loop_w0,
           fc_w1, fc_r_w1, attn_w1, loop_w1):
    src = edge_index[0]
    dst = edge_index[1]
    xp = jnp.pad(x, ((0, NP - N), (0, 0)))

    s3T = _tc_pre(edge_attr, fc_r_w0, fc_r_w1, attn_w0, attn_w1)

    z0, zl0, s12 = _tc0(xp, fc_w0, loop_w0, attn_w0)
    a1_0, a2_0, den0 = _layer_parts(src, dst, s3T[0], s12[0], s12[1],
                                    z0, edge_attr)

    z1, zl1, s12b = _tc1(a1_0, a2_0, den0, fc_r_w0, zl0,
                         fc_w1, loop_w1, attn_w1)
    a1_1, a2_1, den1 = _layer_parts(src, dst, s3T[1], s12b[0], s12b[1],
                                    z1, edge_attr)

    return _tc2(a1_1, a2_1, den1, fc_r_w1, zl1)[:N]


# speedup vs baseline: 11.6362x; 1.0006x over previous
import functools

import jax
import jax.numpy as jnp
from jax import lax
from jax.experimental import pallas as pl
from jax.experimental.pallas import tpu as pltpu
from jax.experimental.pallas import tpu_sc as plsc

N = 10000
E = 320000
D = 128
NP = 10240
NC = 2
NS = 16
EB = E // NS
SCHK = 4000
CH = 80
ROWB = 1024
EBLK = 2560


def _pad8(col0, col1):
    zeros = jnp.zeros((D, 6), jnp.float32)
    return jnp.concatenate([col0, col1, zeros], axis=1)


def _tc_pre_body(ea_ref, fr0_ref, fr1_ref, a0_ref, a1_ref, s3_ref):
    r0 = jnp.dot(fr0_ref[...], a0_ref[2 * D:3 * D, :],
                 preferred_element_type=jnp.float32)
    r1 = jnp.dot(fr1_ref[...], a1_ref[2 * D:3 * D, :],
                 preferred_element_type=jnp.float32)
    R = _pad8(r0, r1)
    s3_ref[...] = lax.dot_general(R, ea_ref[...], (((0,), (1,)), ((), ())),
                                  preferred_element_type=jnp.float32)


def _tc_pre(edge_attr, fc_r_w0, fc_r_w1, attn_w0, attn_w1):
    wfull = pl.BlockSpec((D, D), lambda b: (0, 0))
    afull = pl.BlockSpec((3 * D, 1), lambda b: (0, 0))
    return pl.pallas_call(
        _tc_pre_body,
        grid=(E // EBLK,),
        in_specs=[pl.BlockSpec((EBLK, D), lambda b: (b, 0)),
                  wfull, wfull, afull, afull],
        out_specs=pl.BlockSpec((8, EBLK), lambda b: (0, b)),
        out_shape=jax.ShapeDtypeStruct((8, E), jnp.float32),
    )(edge_attr, fc_r_w0, fc_r_w1, attn_w0, attn_w1)


def _project(z, loopw_ref, aw_ref, z_ref, zl_ref, s12_ref):
    z_ref[...] = z
    zl_ref[...] = jnp.dot(z, loopw_ref[...], preferred_element_type=jnp.float32)
    W = _pad8(aw_ref[0:D, :], aw_ref[D:2 * D, :])
    s12_ref[...] = lax.dot_general(W, z, (((0,), (1,)), ((), ())),
                                   preferred_element_type=jnp.float32)


def _combine(a1_ref, a2_ref, den_ref, frw_ref, zlp_ref):
    den = den_ref[...]
    dens = jnp.where(den > 0.0, den, 1.0)
    msg = (a1_ref[...]
           + jnp.dot(a2_ref[...], frw_ref[...],
                     preferred_element_type=jnp.float32)) / dens
    return jnp.maximum(msg + zlp_ref[...], 0.0)


def _tc0_body(x_ref, fcw_ref, loopw_ref, aw_ref, z_ref, zl_ref, s12_ref):
    z = jnp.dot(x_ref[...], fcw_ref[...], preferred_element_type=jnp.float32)
    _project(z, loopw_ref, aw_ref, z_ref, zl_ref, s12_ref)


def _tc1_body(a1_ref, a2_ref, den_ref, frw_ref, zlp_ref, fcw_ref, loopw_ref,
              aw_ref, z_ref, zl_ref, s12_ref):
    h = _combine(a1_ref, a2_ref, den_ref, frw_ref, zlp_ref)
    z = jnp.dot(h, fcw_ref[...], preferred_element_type=jnp.float32)
    _project(z, loopw_ref, aw_ref, z_ref, zl_ref, s12_ref)


def _tc2_body(a1_ref, a2_ref, den_ref, frw_ref, zlp_ref, out_ref):
    out_ref[...] = _combine(a1_ref, a2_ref, den_ref, frw_ref, zlp_ref)


_ROWBS = pl.BlockSpec((ROWB, D), lambda b: (b, 0))
_DENBS = pl.BlockSpec((ROWB, 1), lambda b: (b, 0))
_WBS = pl.BlockSpec((D, D), lambda b: (0, 0))
_ABS = pl.BlockSpec((3 * D, 1), lambda b: (0, 0))
_PROJ_OUT = dict(
    out_specs=[pl.BlockSpec((ROWB, D), lambda b: (b, 0)),
               pl.BlockSpec((ROWB, D), lambda b: (b, 0)),
               pl.BlockSpec((8, ROWB), lambda b: (0, b))],
    out_shape=[jax.ShapeDtypeStruct((NP, D), jnp.float32),
               jax.ShapeDtypeStruct((NP, D), jnp.float32),
               jax.ShapeDtypeStruct((8, NP), jnp.float32)],
)


def _tc0(x, fc_w, loop_w, attn_w):
    return pl.pallas_call(
        _tc0_body, grid=(NP // ROWB,),
        in_specs=[_ROWBS, _WBS, _WBS, _ABS], **_PROJ_OUT,
    )(x, fc_w, loop_w, attn_w)


def _tc1(a1, a2, den, fc_r_w, zl_prev, fc_w, loop_w, attn_w):
    return pl.pallas_call(
        _tc1_body, grid=(NP // ROWB,),
        in_specs=[_ROWBS, _ROWBS, _DENBS, _WBS, _ROWBS, _WBS, _WBS, _ABS],
        **_PROJ_OUT,
    )(a1, a2, den, fc_r_w, zl_prev, fc_w, loop_w, attn_w)


def _tc2(a1, a2, den, fc_r_w, zl_prev):
    return pl.pallas_call(
        _tc2_body, grid=(NP // ROWB,),
        in_specs=[_ROWBS, _ROWBS, _DENBS, _WBS, _ROWBS],
        out_specs=pl.BlockSpec((ROWB, D), lambda b: (b, 0)),
        out_shape=jax.ShapeDtypeStruct((NP, D), jnp.float32),
    )(a1, a2, den, fc_r_w, zl_prev)


_MESH = plsc.VectorSubcoreMesh(core_axis_name="c", subcore_axis_name="s",
                               num_cores=NC, num_subcores=NS)


@functools.partial(
    pl.kernel,
    out_type=jax.ShapeDtypeStruct((2 * NP + 80, D), jnp.float32),
    mesh=_MESH,
    compiler_params=pltpu.CompilerParams(needs_layout_passes=False),
    scratch_types=[
        pltpu.VMEM((NP,), jnp.float32),
        pltpu.VMEM((NP,), jnp.float32),
        pltpu.VMEM((SCHK,), jnp.int32),
        pltpu.VMEM((SCHK,), jnp.int32),
        pltpu.VMEM((SCHK,), jnp.float32),
        pltpu.VMEM((CH, D), jnp.float32),
        pltpu.VMEM((CH,), jnp.int32),
        pltpu.VMEM((1024,), jnp.float32),
        pltpu.VMEM((8, D), jnp.float32),
        pltpu.VMEM_SHARED((NP, D), jnp.float32),
        pltpu.VMEM_SHARED((NP,), jnp.float32),
    ],
)
def _sc_layer(src_h, dst_h, s3_h, s1_h, s2_h, z_h, ea_h, out_h,
              s1v, s2v, srcv, dstv, exv, zbuf, idxv, dtmp, dtmp2,
              aggsh, dshared):
    c = lax.axis_index("c")
    s = lax.axis_index("s")
    base = s * EB

    pltpu.sync_copy(s1_h, s1v)
    pltpu.sync_copy(s2_h, s2v)

    @pl.loop(0, CH)
    def _zrow(r):
        for q in range(D // 16):
            zbuf[r, pl.ds(q * 16, 16)] = jnp.zeros((16,), jnp.float32)

    @pl.loop(0, 1024 // 16)
    def _zden(i):
        dtmp[pl.ds(i * 16, 16)] = jnp.zeros((16,), jnp.float32)

    for k in range(640 // CH):
        pltpu.sync_copy(zbuf, aggsh.at[pl.ds(s * 640 + k * CH, CH)])

    @pl.when(c == 0)
    def _():
        pltpu.sync_copy(dtmp.at[pl.ds(0, 640)], dshared.at[pl.ds(s * 640, 640)])

    plsc.subcore_barrier()

    @pl.loop(0, EB // SCHK)
    def _super(u):
        sbase = base + u * SCHK
        pltpu.sync_copy(dst_h.at[pl.ds(sbase, SCHK)], dstv)
        pltpu.sync_copy(s3_h.at[pl.ds(sbase, SCHK)], exv)
        pltpu.sync_copy(src_h.at[pl.ds(sbase, SCHK)], srcv)

        @pl.loop(0, SCHK // 16)
        def _logits(i):
            off = i * 16
            sv = srcv[pl.ds(off, 16)]
            dv = dstv[pl.ds(off, 16)]
            g2 = plsc.load_gather(s2v, [dv])
            a = plsc.load_gather(s1v, [sv]) + g2 + exv[pl.ds(off, 16)]
            e = jnp.maximum(a, 0.01 * a)
            cc = jnp.maximum(g2, 0.01 * g2)
            exv[pl.ds(off, 16)] = jnp.exp(e - cc)

        @pl.loop(0, SCHK // CH)
        def _edges(j):
            off = j * CH

            @pl.when(c == 0)
            def _():
                for k in range(CH // 16):
                    idxv[pl.ds(k * 16, 16)] = srcv[pl.ds(off + k * 16, 16)]
                pltpu.sync_copy(z_h.at[idxv], zbuf)

            @pl.when(c == 1)
            def _():
                pltpu.sync_copy(ea_h.at[pl.ds(sbase + off, CH)], zbuf)

            @pl.loop(0, CH // 16)
            def _scale(k):
                exl = exv[pl.ds(off + k * 16, 16)]
                for i in range(16):
                    w = exl[i]
                    e = k * 16 + i
                    for q in range(D // 16):
                        zbuf[e, pl.ds(q * 16, 16)] = (
                            zbuf[e, pl.ds(q * 16, 16)] * w)

            for k in range(CH // 16):
                idxv[pl.ds(k * 16, 16)] = dstv[pl.ds(off + k * 16, 16)]
            pltpu.sync_copy(zbuf, aggsh.at[idxv], add=True)

            @pl.when(c == 0)
            def _():
                pltpu.sync_copy(exv.at[pl.ds(off, CH)], dshared.at[idxv],
                                add=True)

    plsc.subcore_barrier()

    @pl.when(c == 0)
    def _():
        pltpu.sync_copy(aggsh.at[pl.ds(s * 640, 640)],
                        out_h.at[pl.ds(s * 640, 640)])

    @pl.when(c == 1)
    def _():
        pltpu.sync_copy(aggsh.at[pl.ds(s * 640, 640)],
                        out_h.at[pl.ds(NP + s * 640, 640)])

    @pl.when(jnp.logical_and(c == 0, s < 10))
    def _():
        pltpu.sync_copy(dshared.at[pl.ds(s * 1024, 1024)], dtmp)

        @pl.loop(0, 8)
        def _d2(r):
            for q in range(D // 16):
                dtmp2[r, pl.ds(q * 16, 16)] = dtmp[pl.ds(r * 128 + q * 16, 16)]

        pltpu.sync_copy(dtmp2, out_h.at[pl.ds(2 * NP + s * 8, 8)])


def _layer_parts(src, dst, s3row, s1, s2, z, ea):
    agg = _sc_layer(src, dst, s3row, s1, s2, z, ea)
    return agg[:NP], agg[NP:2 * NP], jnp.reshape(agg[2 * NP:], (NP, 1))


def kernel(x, edge_index, edge_attr, fc_w0, fc_r_w0, attn_w0, loop_w0,
           fc_w1, fc_r_w1, attn_w1, loop_w1):
    src = edge_index[0]
    dst = edge_index[1]
    xp = jnp.pad(x, ((0, NP - N), (0, 0)))

    s3T = _tc_pre(edge_attr, fc_r_w0, fc_r_w1, attn_w0, attn_w1)

    z0, zl0, s12 = _tc0(xp, fc_w0, loop_w0, attn_w0)
    a1_0, a2_0, den0 = _layer_parts(src, dst, s3T[0], s12[0], s12[1],
                                    z0, edge_attr)

    z1, zl1, s12b = _tc1(a1_0, a2_0, den0, fc_r_w0, zl0,
                         fc_w1, loop_w1, attn_w1)
    a1_1, a2_1, den1 = _layer_parts(src, dst, s3T[1], s12b[0], s12b[1],
                                    z1, edge_attr)

    return _tc2(a1_1, a2_1, den1, fc_r_w1, zl1)[:N]


# speedup vs baseline: 15.6203x; 1.3424x over previous
import functools

import jax
import jax.numpy as jnp
from jax import lax
from jax.experimental import pallas as pl
from jax.experimental.pallas import tpu as pltpu
from jax.experimental.pallas import tpu_sc as plsc

N = 10000
E = 320000
D = 128
NP = 10240
NC = 2
NS = 16
EB = E // NS
SCHK = 800
CH = 80
ROWB = 1024
EBLK = 2560


def _pad8(col0, col1):
    zeros = jnp.zeros((D, 6), jnp.float32)
    return jnp.concatenate([col0, col1, zeros], axis=1)


def _tc_pre_body(ea_ref, fr0_ref, fr1_ref, a0_ref, a1_ref, s3_ref):
    r0 = jnp.dot(fr0_ref[...], a0_ref[2 * D:3 * D, :],
                 preferred_element_type=jnp.float32)
    r1 = jnp.dot(fr1_ref[...], a1_ref[2 * D:3 * D, :],
                 preferred_element_type=jnp.float32)
    R = _pad8(r0, r1)
    s3_ref[...] = lax.dot_general(R, ea_ref[...], (((0,), (1,)), ((), ())),
                                  preferred_element_type=jnp.float32)


def _tc_pre(edge_attr, fc_r_w0, fc_r_w1, attn_w0, attn_w1):
    wfull = pl.BlockSpec((D, D), lambda b: (0, 0))
    afull = pl.BlockSpec((3 * D, 1), lambda b: (0, 0))
    return pl.pallas_call(
        _tc_pre_body,
        grid=(E // EBLK,),
        in_specs=[pl.BlockSpec((EBLK, D), lambda b: (b, 0)),
                  wfull, wfull, afull, afull],
        out_specs=pl.BlockSpec((8, EBLK), lambda b: (0, b)),
        out_shape=jax.ShapeDtypeStruct((8, E), jnp.float32),
    )(edge_attr, fc_r_w0, fc_r_w1, attn_w0, attn_w1)


def _project(z, loopw_ref, aw_ref, z_ref, zl_ref, s12_ref):
    z_ref[...] = z
    zl_ref[...] = jnp.dot(z, loopw_ref[...], preferred_element_type=jnp.float32)
    W = _pad8(aw_ref[0:D, :], aw_ref[D:2 * D, :])
    s12_ref[...] = lax.dot_general(W, z, (((0,), (1,)), ((), ())),
                                   preferred_element_type=jnp.float32)


def _combine(a1_ref, a2_ref, den_ref, frw_ref, zlp_ref):
    den = den_ref[...]
    dens = jnp.where(den > 0.0, den, 1.0)
    msg = (a1_ref[...]
           + jnp.dot(a2_ref[...], frw_ref[...],
                     preferred_element_type=jnp.float32)) / dens
    return jnp.maximum(msg + zlp_ref[...], 0.0)


def _tc0_body(x_ref, fcw_ref, loopw_ref, aw_ref, z_ref, zl_ref, s12_ref):
    z = jnp.dot(x_ref[...], fcw_ref[...], preferred_element_type=jnp.float32)
    _project(z, loopw_ref, aw_ref, z_ref, zl_ref, s12_ref)


def _tc1_body(a1_ref, a2_ref, den_ref, frw_ref, zlp_ref, fcw_ref, loopw_ref,
              aw_ref, z_ref, zl_ref, s12_ref):
    h = _combine(a1_ref, a2_ref, den_ref, frw_ref, zlp_ref)
    z = jnp.dot(h, fcw_ref[...], preferred_element_type=jnp.float32)
    _project(z, loopw_ref, aw_ref, z_ref, zl_ref, s12_ref)


def _tc2_body(a1_ref, a2_ref, den_ref, frw_ref, zlp_ref, out_ref):
    out_ref[...] = _combine(a1_ref, a2_ref, den_ref, frw_ref, zlp_ref)


_ROWBS = pl.BlockSpec((ROWB, D), lambda b: (b, 0))
_DENBS = pl.BlockSpec((ROWB, 1), lambda b: (b, 0))
_WBS = pl.BlockSpec((D, D), lambda b: (0, 0))
_ABS = pl.BlockSpec((3 * D, 1), lambda b: (0, 0))
_PROJ_OUT = dict(
    out_specs=[pl.BlockSpec((ROWB, D), lambda b: (b, 0)),
               pl.BlockSpec((ROWB, D), lambda b: (b, 0)),
               pl.BlockSpec((8, ROWB), lambda b: (0, b))],
    out_shape=[jax.ShapeDtypeStruct((NP, D), jnp.float32),
               jax.ShapeDtypeStruct((NP, D), jnp.float32),
               jax.ShapeDtypeStruct((8, NP), jnp.float32)],
)


def _tc0(x, fc_w, loop_w, attn_w):
    return pl.pallas_call(
        _tc0_body, grid=(NP // ROWB,),
        in_specs=[_ROWBS, _WBS, _WBS, _ABS], **_PROJ_OUT,
    )(x, fc_w, loop_w, attn_w)


def _tc1(a1, a2, den, fc_r_w, zl_prev, fc_w, loop_w, attn_w):
    return pl.pallas_call(
        _tc1_body, grid=(NP // ROWB,),
        in_specs=[_ROWBS, _ROWBS, _DENBS, _WBS, _ROWBS, _WBS, _WBS, _ABS],
        **_PROJ_OUT,
    )(a1, a2, den, fc_r_w, zl_prev, fc_w, loop_w, attn_w)


def _tc2(a1, a2, den, fc_r_w, zl_prev):
    return pl.pallas_call(
        _tc2_body, grid=(NP // ROWB,),
        in_specs=[_ROWBS, _ROWBS, _DENBS, _WBS, _ROWBS],
        out_specs=pl.BlockSpec((ROWB, D), lambda b: (b, 0)),
        out_shape=jax.ShapeDtypeStruct((NP, D), jnp.float32),
    )(a1, a2, den, fc_r_w, zl_prev)


_MESH = plsc.VectorSubcoreMesh(core_axis_name="c", subcore_axis_name="s",
                               num_cores=NC, num_subcores=NS)


@functools.partial(
    pl.kernel,
    out_type=jax.ShapeDtypeStruct((2 * NP + 80, D), jnp.float32),
    mesh=_MESH,
    compiler_params=pltpu.CompilerParams(needs_layout_passes=False),
    scratch_types=[
        pltpu.VMEM((NP,), jnp.float32),
        pltpu.VMEM((NP,), jnp.float32),
        pltpu.VMEM((SCHK,), jnp.int32),
        pltpu.VMEM((SCHK,), jnp.int32),
        pltpu.VMEM((SCHK,), jnp.float32),
        pltpu.VMEM((CH, D), jnp.float32),
        pltpu.VMEM((CH, D), jnp.float32),
        pltpu.VMEM((CH,), jnp.int32),
        pltpu.VMEM((CH,), jnp.int32),
        pltpu.VMEM((CH,), jnp.int32),
        pltpu.VMEM((CH,), jnp.int32),
        pltpu.VMEM((1024,), jnp.float32),
        pltpu.VMEM((8, D), jnp.float32),
        pltpu.VMEM_SHARED((NP, D), jnp.float32),
        pltpu.VMEM_SHARED((NP,), jnp.float32),
        pltpu.SemaphoreType.DMA,
        pltpu.SemaphoreType.DMA,
        pltpu.SemaphoreType.DMA,
        pltpu.SemaphoreType.DMA,
        pltpu.SemaphoreType.DMA,
        pltpu.SemaphoreType.DMA,
    ],
)
def _sc_layer(src_h, dst_h, s3_h, s1_h, s2_h, z_h, ea_h, out_h,
              s1v, s2v, srcv, dstv, exv, zb0, zb1,
              idxg0, idxg1, idxd0, idxd1, dtmp, dtmp2, aggsh, dshared,
              semg0, semg1, sems0, sems1, seme0, seme1):
    c = lax.axis_index("c")
    s = lax.axis_index("s")
    base = s * EB
    zb = (zb0, zb1)
    idxg = (idxg0, idxg1)
    idxd = (idxd0, idxd1)
    semg = (semg0, semg1)
    sems = (sems0, sems1)
    seme = (seme0, seme1)
    CQ = SCHK // CH

    pltpu.sync_copy(s1_h, s1v)
    pltpu.sync_copy(s2_h, s2v)

    @pl.loop(0, CH)
    def _zrow(r):
        for q in range(D // 16):
            zb0[r, pl.ds(q * 16, 16)] = jnp.zeros((16,), jnp.float32)

    @pl.loop(0, 1024 // 16)
    def _zden(i):
        dtmp[pl.ds(i * 16, 16)] = jnp.zeros((16,), jnp.float32)

    for k in range(640 // CH):
        pltpu.sync_copy(zb0, aggsh.at[pl.ds(s * 640 + k * CH, CH)])

    @pl.when(c == 0)
    def _():
        pltpu.sync_copy(dtmp.at[pl.ds(0, 640)], dshared.at[pl.ds(s * 640, 640)])

    plsc.subcore_barrier()

    @pl.loop(0, EB // SCHK)
    def _super(u):
        sbase = base + u * SCHK
        pltpu.sync_copy(dst_h.at[pl.ds(sbase, SCHK)], dstv)
        pltpu.sync_copy(s3_h.at[pl.ds(sbase, SCHK)], exv)
        pltpu.sync_copy(src_h.at[pl.ds(sbase, SCHK)], srcv)

        @pl.loop(0, SCHK // 16)
        def _logits(i):
            off = i * 16
            sv = srcv[pl.ds(off, 16)]
            dv = dstv[pl.ds(off, 16)]
            g2 = plsc.load_gather(s2v, [dv])
            a = plsc.load_gather(s1v, [sv]) + g2 + exv[pl.ds(off, 16)]
            e = jnp.maximum(a, 0.01 * a)
            cc = jnp.maximum(g2, 0.01 * g2)
            exv[pl.ds(off, 16)] = jnp.exp(e - cc)

        def _stage_gather(p, co):
            @pl.when(c == 0)
            def _():
                for k in range(CH // 16):
                    idxg[p][pl.ds(k * 16, 16)] = srcv[pl.ds(co + k * 16, 16)]
                pltpu.async_copy(z_h.at[idxg[p]], zb[p], semg[p])

            @pl.when(c == 1)
            def _():
                pltpu.async_copy(ea_h.at[pl.ds(sbase + co, CH)], zb[p],
                                 semg[p])

        def _wait_gather(p):
            pltpu.make_async_copy(ea_h.at[pl.ds(sbase, CH)], zb[p],
                                  semg[p]).wait()

        def _scale(p, co):
            @pl.loop(0, CH // 16)
            def _sc(k):
                exl = exv[pl.ds(co + k * 16, 16)]
                for i in range(16):
                    w = exl[i]
                    e = k * 16 + i
                    for q in range(D // 16):
                        zb[p][e, pl.ds(q * 16, 16)] = (
                            zb[p][e, pl.ds(q * 16, 16)] * w)

        def _start_scatter(p, co):
            for k in range(CH // 16):
                idxd[p][pl.ds(k * 16, 16)] = dstv[pl.ds(co + k * 16, 16)]
            pltpu.async_copy(zb[p], aggsh.at[idxd[p]], sems[p], add=True)

            @pl.when(c == 0)
            def _():
                pltpu.async_copy(exv.at[pl.ds(co, CH)], dshared.at[idxd[p]],
                                 seme[p], add=True)

        def _wait_scatter(p):
            pltpu.make_async_copy(zb[p], aggsh.at[idxd[p]], sems[p]).wait()

            @pl.when(c == 0)
            def _():
                pltpu.make_async_copy(exv.at[pl.ds(0, CH)],
                                      dshared.at[idxd[p]], seme[p]).wait()

        _stage_gather(0, 0)

        @pl.loop(0, CQ // 2)
        def _pair(t):
            coa = 2 * t * CH
            cob = coa + CH
            _wait_gather(0)

            @pl.when(t > 0)
            def _():
                _wait_scatter(1)

            _stage_gather(1, cob)
            _scale(0, coa)
            _start_scatter(0, coa)
            _wait_gather(1)

            @pl.when(t < CQ // 2 - 1)
            def _():
                _wait_scatter(0)
                _stage_gather(0, coa + 2 * CH)

            _scale(1, cob)
            _start_scatter(1, cob)

        _wait_scatter(0)
        _wait_scatter(1)

    plsc.subcore_barrier()

    @pl.when(c == 0)
    def _():
        pltpu.sync_copy(aggsh.at[pl.ds(s * 640, 640)],
                        out_h.at[pl.ds(s * 640, 640)])

    @pl.when(c == 1)
    def _():
        pltpu.sync_copy(aggsh.at[pl.ds(s * 640, 640)],
                        out_h.at[pl.ds(NP + s * 640, 640)])

    @pl.when(jnp.logical_and(c == 0, s < 10))
    def _():
        pltpu.sync_copy(dshared.at[pl.ds(s * 1024, 1024)], dtmp)

        @pl.loop(0, 8)
        def _d2(r):
            for q in range(D // 16):
                dtmp2[r, pl.ds(q * 16, 16)] = dtmp[pl.ds(r * 128 + q * 16, 16)]

        pltpu.sync_copy(dtmp2, out_h.at[pl.ds(2 * NP + s * 8, 8)])


def _layer_parts(src, dst, s3row, s1, s2, z, ea):
    agg = _sc_layer(src, dst, s3row, s1, s2, z, ea)
    return agg[:NP], agg[NP:2 * NP], jnp.reshape(agg[2 * NP:], (NP, 1))


def kernel(x, edge_index, edge_attr, fc_w0, fc_r_w0, attn_w0, loop_w0,
           fc_w1, fc_r_w1, attn_w1, loop_w1):
    src = edge_index[0]
    dst = edge_index[1]
    xp = jnp.pad(x, ((0, NP - N), (0, 0)))

    s3T = _tc_pre(edge_attr, fc_r_w0, fc_r_w1, attn_w0, attn_w1)

    z0, zl0, s12 = _tc0(xp, fc_w0, loop_w0, attn_w0)
    a1_0, a2_0, den0 = _layer_parts(src, dst, s3T[0], s12[0], s12[1],
                                    z0, edge_attr)

    z1, zl1, s12b = _tc1(a1_0, a2_0, den0, fc_r_w0, zl0,
                         fc_w1, loop_w1, attn_w1)
    a1_1, a2_1, den1 = _layer_parts(src, dst, s3T[1], s12b[0], s12b[1],
                                    z1, edge_attr)

    return _tc2(a1_1, a2_1, den1, fc_r_w1, zl1)[:N]


# speedup vs baseline: 16.6385x; 1.0652x over previous
import functools

import jax
import jax.numpy as jnp
from jax import lax
from jax.experimental import pallas as pl
from jax.experimental.pallas import tpu as pltpu
from jax.experimental.pallas import tpu_sc as plsc

N = 10000
E = 320000
D = 128
NP = 10240
NC = 2
NS = 16
EB = E // NS
SCHK = 800
CH = 80
ROWB = 1024
EBLK = 2560


def _pad8(col0, col1):
    zeros = jnp.zeros((D, 6), jnp.float32)
    return jnp.concatenate([col0, col1, zeros], axis=1)


def _tc_pre_body(ea_ref, fr0_ref, fr1_ref, a0_ref, a1_ref, s3_ref):
    r0 = jnp.dot(fr0_ref[...], a0_ref[2 * D:3 * D, :],
                 preferred_element_type=jnp.float32)
    r1 = jnp.dot(fr1_ref[...], a1_ref[2 * D:3 * D, :],
                 preferred_element_type=jnp.float32)
    R = _pad8(r0, r1)
    s3_ref[...] = lax.dot_general(R, ea_ref[...], (((0,), (1,)), ((), ())),
                                  preferred_element_type=jnp.float32)


def _tc_pre(edge_attr, fc_r_w0, fc_r_w1, attn_w0, attn_w1):
    wfull = pl.BlockSpec((D, D), lambda b: (0, 0))
    afull = pl.BlockSpec((3 * D, 1), lambda b: (0, 0))
    return pl.pallas_call(
        _tc_pre_body,
        grid=(E // EBLK,),
        in_specs=[pl.BlockSpec((EBLK, D), lambda b: (b, 0)),
                  wfull, wfull, afull, afull],
        out_specs=pl.BlockSpec((8, EBLK), lambda b: (0, b)),
        out_shape=jax.ShapeDtypeStruct((8, E), jnp.float32),
    )(edge_attr, fc_r_w0, fc_r_w1, attn_w0, attn_w1)


def _project(z, loopw_ref, aw_ref, z_ref, zl_ref, s12_ref):
    z_ref[...] = z
    zl_ref[...] = jnp.dot(z, loopw_ref[...], preferred_element_type=jnp.float32)
    W = _pad8(aw_ref[0:D, :], aw_ref[D:2 * D, :])
    s12_ref[...] = lax.dot_general(W, z, (((0,), (1,)), ((), ())),
                                   preferred_element_type=jnp.float32)


def _combine(a1_ref, a2_ref, den_ref, frw_ref, zlp_ref):
    den = den_ref[...]
    dens = jnp.where(den > 0.0, den, 1.0)
    msg = (a1_ref[...]
           + jnp.dot(a2_ref[...], frw_ref[...],
                     preferred_element_type=jnp.float32)) / dens
    return jnp.maximum(msg + zlp_ref[...], 0.0)


def _tc0_body(x_ref, fcw_ref, loopw_ref, aw_ref, z_ref, zl_ref, s12_ref):
    z = jnp.dot(x_ref[...], fcw_ref[...], preferred_element_type=jnp.float32)
    _project(z, loopw_ref, aw_ref, z_ref, zl_ref, s12_ref)


def _tc1_body(a1_ref, a2_ref, den_ref, frw_ref, zlp_ref, fcw_ref, loopw_ref,
              aw_ref, z_ref, zl_ref, s12_ref):
    h = _combine(a1_ref, a2_ref, den_ref, frw_ref, zlp_ref)
    z = jnp.dot(h, fcw_ref[...], preferred_element_type=jnp.float32)
    _project(z, loopw_ref, aw_ref, z_ref, zl_ref, s12_ref)


def _tc2_body(a1_ref, a2_ref, den_ref, frw_ref, zlp_ref, out_ref):
    out_ref[...] = _combine(a1_ref, a2_ref, den_ref, frw_ref, zlp_ref)


_ROWBS = pl.BlockSpec((ROWB, D), lambda b: (b, 0))
_DENBS = pl.BlockSpec((ROWB, 1), lambda b: (b, 0))
_WBS = pl.BlockSpec((D, D), lambda b: (0, 0))
_ABS = pl.BlockSpec((3 * D, 1), lambda b: (0, 0))
_PROJ_OUT = dict(
    out_specs=[pl.BlockSpec((ROWB, D), lambda b: (b, 0)),
               pl.BlockSpec((ROWB, D), lambda b: (b, 0)),
               pl.BlockSpec((8, ROWB), lambda b: (0, b))],
    out_shape=[jax.ShapeDtypeStruct((NP, D), jnp.float32),
               jax.ShapeDtypeStruct((NP, D), jnp.float32),
               jax.ShapeDtypeStruct((8, NP), jnp.float32)],
)


def _tc0(x, fc_w, loop_w, attn_w):
    return pl.pallas_call(
        _tc0_body, grid=(NP // ROWB,),
        in_specs=[_ROWBS, _WBS, _WBS, _ABS], **_PROJ_OUT,
    )(x, fc_w, loop_w, attn_w)


def _tc1(a1, a2, den, fc_r_w, zl_prev, fc_w, loop_w, attn_w):
    return pl.pallas_call(
        _tc1_body, grid=(NP // ROWB,),
        in_specs=[_ROWBS, _ROWBS, _DENBS, _WBS, _ROWBS, _WBS, _WBS, _ABS],
        **_PROJ_OUT,
    )(a1, a2, den, fc_r_w, zl_prev, fc_w, loop_w, attn_w)


def _tc2(a1, a2, den, fc_r_w, zl_prev):
    return pl.pallas_call(
        _tc2_body, grid=(NP // ROWB,),
        in_specs=[_ROWBS, _ROWBS, _DENBS, _WBS, _ROWBS],
        out_specs=pl.BlockSpec((ROWB, D), lambda b: (b, 0)),
        out_shape=jax.ShapeDtypeStruct((NP, D), jnp.float32),
    )(a1, a2, den, fc_r_w, zl_prev)


_MESH = plsc.VectorSubcoreMesh(core_axis_name="c", subcore_axis_name="s",
                               num_cores=NC, num_subcores=NS)


@functools.partial(
    pl.kernel,
    out_type=jax.ShapeDtypeStruct((2 * NP + 80, D), jnp.float32),
    mesh=_MESH,
    compiler_params=pltpu.CompilerParams(needs_layout_passes=False),
    scratch_types=[
        pltpu.VMEM((NP,), jnp.float32),
        pltpu.VMEM((NP,), jnp.float32),
        pltpu.VMEM((3, SCHK), jnp.int32),
        pltpu.VMEM((SCHK,), jnp.float32),
        pltpu.VMEM((CH, D), jnp.float32),
        pltpu.VMEM((CH, D), jnp.float32),
        pltpu.VMEM((CH,), jnp.int32),
        pltpu.VMEM((CH,), jnp.int32),
        pltpu.VMEM((CH,), jnp.int32),
        pltpu.VMEM((CH,), jnp.int32),
        pltpu.VMEM((1024,), jnp.float32),
        pltpu.VMEM((8, D), jnp.float32),
        pltpu.VMEM_SHARED((NP, D), jnp.float32),
        pltpu.VMEM_SHARED((NP,), jnp.float32),
        pltpu.SemaphoreType.DMA,
        pltpu.SemaphoreType.DMA,
        pltpu.SemaphoreType.DMA,
        pltpu.SemaphoreType.DMA,
        pltpu.SemaphoreType.DMA,
        pltpu.SemaphoreType.DMA,
    ],
)
def _sc_layer(epk_h, s1_h, s2_h, z_h, ea_h, out_h,
              s1v, s2v, esv, exv, zb0, zb1,
              idxg0, idxg1, idxd0, idxd1, dtmp, dtmp2, aggsh, dshared,
              semg0, semg1, sems0, sems1, seme0, seme1):
    c = lax.axis_index("c")
    s = lax.axis_index("s")
    base = s * EB
    zb = (zb0, zb1)
    idxg = (idxg0, idxg1)
    idxd = (idxd0, idxd1)
    semg = (semg0, semg1)
    sems = (sems0, sems1)
    seme = (seme0, seme1)
    CQ = SCHK // CH

    pltpu.sync_copy(s1_h, s1v)
    pltpu.sync_copy(s2_h, s2v)

    @pl.loop(0, CH)
    def _zrow(r):
        for q in range(D // 16):
            zb0[r, pl.ds(q * 16, 16)] = jnp.zeros((16,), jnp.float32)

    @pl.loop(0, 1024 // 16)
    def _zden(i):
        dtmp[pl.ds(i * 16, 16)] = jnp.zeros((16,), jnp.float32)

    for k in range(640 // CH):
        pltpu.sync_copy(zb0, aggsh.at[pl.ds(s * 640 + k * CH, CH)])

    @pl.when(c == 0)
    def _():
        pltpu.sync_copy(dtmp.at[pl.ds(0, 640)], dshared.at[pl.ds(s * 640, 640)])

    plsc.subcore_barrier()

    @pl.loop(0, EB // SCHK)
    def _super(u):
        sbase = base + u * SCHK
        pltpu.sync_copy(epk_h.at[s * (EB // SCHK) + u], esv)

        def _stage_gather(p, co):
            @pl.when(c == 0)
            def _():
                for k in range(CH // 16):
                    idxg[p][pl.ds(k * 16, 16)] = esv[0, pl.ds(co + k * 16, 16)]
                pltpu.async_copy(z_h.at[idxg[p]], zb[p], semg[p])

            @pl.when(c == 1)
            def _():
                pltpu.async_copy(ea_h.at[pl.ds(sbase + co, CH)], zb[p],
                                 semg[p])

        def _wait_gather(p):
            pltpu.make_async_copy(ea_h.at[pl.ds(sbase, CH)], zb[p],
                                  semg[p]).wait()

        def _scale(p, co):
            @pl.loop(0, CH // 16)
            def _sc(k):
                o = co + k * 16
                sv = esv[0, pl.ds(o, 16)]
                dv = esv[1, pl.ds(o, 16)]
                s3 = plsc.bitcast(esv[2, pl.ds(o, 16)], jnp.float32)
                g2 = plsc.load_gather(s2v, [dv])
                a = plsc.load_gather(s1v, [sv]) + g2 + s3
                e = jnp.maximum(a, 0.01 * a)
                cc = jnp.maximum(g2, 0.01 * g2)
                exl = jnp.exp(e - cc)
                exv[pl.ds(o, 16)] = exl
                for i in range(16):
                    w = exl[i]
                    e2 = k * 16 + i
                    for q in range(D // 16):
                        zb[p][e2, pl.ds(q * 16, 16)] = (
                            zb[p][e2, pl.ds(q * 16, 16)] * w)

        def _start_scatter(p, co):
            for k in range(CH // 16):
                idxd[p][pl.ds(k * 16, 16)] = esv[1, pl.ds(co + k * 16, 16)]
            pltpu.async_copy(zb[p], aggsh.at[idxd[p]], sems[p], add=True)

            @pl.when(c == 0)
            def _():
                pltpu.async_copy(exv.at[pl.ds(co, CH)], dshared.at[idxd[p]],
                                 seme[p], add=True)

        def _wait_scatter(p):
            pltpu.make_async_copy(zb[p], aggsh.at[idxd[p]], sems[p]).wait()

            @pl.when(c == 0)
            def _():
                pltpu.make_async_copy(exv.at[pl.ds(0, CH)],
                                      dshared.at[idxd[p]], seme[p]).wait()

        _stage_gather(0, 0)

        @pl.loop(0, CQ // 2)
        def _pair(t):
            coa = 2 * t * CH
            cob = coa + CH
            _wait_gather(0)

            @pl.when(t > 0)
            def _():
                _wait_scatter(1)

            _stage_gather(1, cob)
            _scale(0, coa)
            _start_scatter(0, coa)
            _wait_gather(1)

            @pl.when(t < CQ // 2 - 1)
            def _():
                _wait_scatter(0)
                _stage_gather(0, coa + 2 * CH)

            _scale(1, cob)
            _start_scatter(1, cob)

        _wait_scatter(0)
        _wait_scatter(1)

    plsc.subcore_barrier()

    @pl.when(c == 0)
    def _():
        pltpu.sync_copy(aggsh.at[pl.ds(s * 640, 640)],
                        out_h.at[pl.ds(s * 640, 640)])

    @pl.when(c == 1)
    def _():
        pltpu.sync_copy(aggsh.at[pl.ds(s * 640, 640)],
                        out_h.at[pl.ds(NP + s * 640, 640)])

    @pl.when(jnp.logical_and(c == 0, s < 10))
    def _():
        pltpu.sync_copy(dshared.at[pl.ds(s * 1024, 1024)], dtmp)

        @pl.loop(0, 8)
        def _d2(r):
            for q in range(D // 16):
                dtmp2[r, pl.ds(q * 16, 16)] = dtmp[pl.ds(r * 128 + q * 16, 16)]

        pltpu.sync_copy(dtmp2, out_h.at[pl.ds(2 * NP + s * 8, 8)])


def _pack_edges(src, dst, s3row):
    nblk = E // SCHK
    s3b = jax.lax.bitcast_convert_type(s3row, jnp.int32)
    return jnp.stack([jnp.reshape(src, (nblk, SCHK)),
                      jnp.reshape(dst, (nblk, SCHK)),
                      jnp.reshape(s3b, (nblk, SCHK))], axis=1)


def _layer_parts(epk, s1, s2, z, ea):
    agg = _sc_layer(epk, s1, s2, z, ea)
    return agg[:NP], agg[NP:2 * NP], jnp.reshape(agg[2 * NP:], (NP, 1))


def kernel(x, edge_index, edge_attr, fc_w0, fc_r_w0, attn_w0, loop_w0,
           fc_w1, fc_r_w1, attn_w1, loop_w1):
    src = edge_index[0]
    dst = edge_index[1]
    xp = jnp.pad(x, ((0, NP - N), (0, 0)))

    s3T = _tc_pre(edge_attr, fc_r_w0, fc_r_w1, attn_w0, attn_w1)
    epk0 = _pack_edges(src, dst, s3T[0])
    epk1 = _pack_edges(src, dst, s3T[1])

    z0, zl0, s12 = _tc0(xp, fc_w0, loop_w0, attn_w0)
    a1_0, a2_0, den0 = _layer_parts(epk0, s12[0], s12[1], z0, edge_attr)

    z1, zl1, s12b = _tc1(a1_0, a2_0, den0, fc_r_w0, zl0,
                         fc_w1, loop_w1, attn_w1)
    a1_1, a2_1, den1 = _layer_parts(epk1, s12b[0], s12b[1], z1, edge_attr)

    return _tc2(a1_1, a2_1, den1, fc_r_w1, zl1)[:N]


# speedup vs baseline: 16.9228x; 1.0171x over previous
import functools

import jax
import jax.numpy as jnp
from jax import lax
from jax.experimental import pallas as pl
from jax.experimental.pallas import tpu as pltpu
from jax.experimental.pallas import tpu_sc as plsc

N = 10000
E = 320000
D = 128
NP = 10240
NC = 2
NS = 16
EB = E // NS
SCHK = 800
CH = 80
ROWB = 1024
EBLK = 2560


def _pad8(col0, col1):
    zeros = jnp.zeros((D, 6), jnp.float32)
    return jnp.concatenate([col0, col1, zeros], axis=1)


def _tc_pre_body(ea_ref, fr0_ref, fr1_ref, a0_ref, a1_ref, s3_ref):
    r0 = jnp.dot(fr0_ref[...], a0_ref[2 * D:3 * D, :],
                 preferred_element_type=jnp.float32)
    r1 = jnp.dot(fr1_ref[...], a1_ref[2 * D:3 * D, :],
                 preferred_element_type=jnp.float32)
    R = _pad8(r0, r1)
    s3_ref[...] = lax.dot_general(R, ea_ref[...], (((0,), (1,)), ((), ())),
                                  preferred_element_type=jnp.float32)


def _tc_pre(edge_attr, fc_r_w0, fc_r_w1, attn_w0, attn_w1):
    wfull = pl.BlockSpec((D, D), lambda b: (0, 0))
    afull = pl.BlockSpec((3 * D, 1), lambda b: (0, 0))
    return pl.pallas_call(
        _tc_pre_body,
        grid=(E // EBLK,),
        in_specs=[pl.BlockSpec((EBLK, D), lambda b: (b, 0)),
                  wfull, wfull, afull, afull],
        out_specs=pl.BlockSpec((8, EBLK), lambda b: (0, b)),
        out_shape=jax.ShapeDtypeStruct((8, E), jnp.float32),
    )(edge_attr, fc_r_w0, fc_r_w1, attn_w0, attn_w1)


def _project(z, loopw_ref, aw_ref, z_ref, zl_ref, s12_ref):
    z_ref[...] = z
    zl_ref[...] = jnp.dot(z, loopw_ref[...], preferred_element_type=jnp.float32)
    W = _pad8(aw_ref[0:D, :], aw_ref[D:2 * D, :])
    s12_ref[...] = lax.dot_general(W, z, (((0,), (1,)), ((), ())),
                                   preferred_element_type=jnp.float32)


def _combine(a1_ref, a2_ref, den_ref, frw_ref, zlp_ref):
    den = den_ref[...]
    dens = jnp.where(den > 0.0, den, 1.0)
    msg = (a1_ref[...]
           + jnp.dot(a2_ref[...], frw_ref[...],
                     preferred_element_type=jnp.float32)) / dens
    return jnp.maximum(msg + zlp_ref[...], 0.0)


def _tc0_body(x_ref, fcw_ref, loopw_ref, aw_ref, z_ref, zl_ref, s12_ref):
    z = jnp.dot(x_ref[...], fcw_ref[...], preferred_element_type=jnp.float32)
    _project(z, loopw_ref, aw_ref, z_ref, zl_ref, s12_ref)


def _tc1_body(a1_ref, a2_ref, den_ref, frw_ref, zlp_ref, fcw_ref, loopw_ref,
              aw_ref, z_ref, zl_ref, s12_ref):
    h = _combine(a1_ref, a2_ref, den_ref, frw_ref, zlp_ref)
    z = jnp.dot(h, fcw_ref[...], preferred_element_type=jnp.float32)
    _project(z, loopw_ref, aw_ref, z_ref, zl_ref, s12_ref)


def _tc2_body(a1_ref, a2_ref, den_ref, frw_ref, zlp_ref, out_ref):
    out_ref[...] = _combine(a1_ref, a2_ref, den_ref, frw_ref, zlp_ref)


_ROWBS = pl.BlockSpec((ROWB, D), lambda b: (b, 0))
_A1BS = pl.BlockSpec((ROWB, D), lambda b: (b, 0))
_A2BS = pl.BlockSpec((ROWB, D), lambda b: (NP // ROWB + b, 0))
_DENBS = pl.BlockSpec((ROWB, 1), lambda b: (b, 0))
_WBS = pl.BlockSpec((D, D), lambda b: (0, 0))
_ABS = pl.BlockSpec((3 * D, 1), lambda b: (0, 0))
_PROJ_OUT = dict(
    out_specs=[pl.BlockSpec((ROWB, D), lambda b: (b, 0)),
               pl.BlockSpec((ROWB, D), lambda b: (b, 0)),
               pl.BlockSpec((8, ROWB), lambda b: (0, b))],
    out_shape=[jax.ShapeDtypeStruct((NP, D), jnp.float32),
               jax.ShapeDtypeStruct((NP, D), jnp.float32),
               jax.ShapeDtypeStruct((8, NP), jnp.float32)],
)


def _tc0(x, fc_w, loop_w, attn_w):
    return pl.pallas_call(
        _tc0_body, grid=(NP // ROWB,),
        in_specs=[_ROWBS, _WBS, _WBS, _ABS], **_PROJ_OUT,
    )(x, fc_w, loop_w, attn_w)


def _tc1(agg, den, fc_r_w, zl_prev, fc_w, loop_w, attn_w):
    return pl.pallas_call(
        _tc1_body, grid=(NP // ROWB,),
        in_specs=[_A1BS, _A2BS, _DENBS, _WBS, _ROWBS, _WBS, _WBS, _ABS],
        **_PROJ_OUT,
    )(agg, agg, den, fc_r_w, zl_prev, fc_w, loop_w, attn_w)


def _tc2(agg, den, fc_r_w, zl_prev):
    return pl.pallas_call(
        _tc2_body, grid=(NP // ROWB,),
        in_specs=[_A1BS, _A2BS, _DENBS, _WBS, _ROWBS],
        out_specs=pl.BlockSpec((ROWB, D), lambda b: (b, 0)),
        out_shape=jax.ShapeDtypeStruct((NP, D), jnp.float32),
    )(agg, agg, den, fc_r_w, zl_prev)


_MESH = plsc.VectorSubcoreMesh(core_axis_name="c", subcore_axis_name="s",
                               num_cores=NC, num_subcores=NS)


@functools.partial(
    pl.kernel,
    out_type=jax.ShapeDtypeStruct((2 * NP + 80, D), jnp.float32),
    mesh=_MESH,
    compiler_params=pltpu.CompilerParams(needs_layout_passes=False),
    scratch_types=[
        pltpu.VMEM((NP,), jnp.float32),
        pltpu.VMEM((NP,), jnp.float32),
        pltpu.VMEM((3, SCHK), jnp.int32),
        pltpu.VMEM((SCHK,), jnp.float32),
        pltpu.VMEM((CH, D), jnp.float32),
        pltpu.VMEM((CH, D), jnp.float32),
        pltpu.VMEM((CH,), jnp.int32),
        pltpu.VMEM((CH,), jnp.int32),
        pltpu.VMEM((CH,), jnp.int32),
        pltpu.VMEM((CH,), jnp.int32),
        pltpu.VMEM((1024,), jnp.float32),
        pltpu.VMEM((8, D), jnp.float32),
        pltpu.VMEM_SHARED((NP, D), jnp.float32),
        pltpu.VMEM_SHARED((NP,), jnp.float32),
        pltpu.SemaphoreType.DMA,
        pltpu.SemaphoreType.DMA,
        pltpu.SemaphoreType.DMA,
        pltpu.SemaphoreType.DMA,
        pltpu.SemaphoreType.DMA,
        pltpu.SemaphoreType.DMA,
    ],
)
def _sc_layer(epk_h, s1_h, s2_h, z_h, ea_h, out_h,
              s1v, s2v, esv, exv, zb0, zb1,
              idxg0, idxg1, idxd0, idxd1, dtmp, dtmp2, aggsh, dshared,
              semg0, semg1, sems0, sems1, seme0, seme1):
    c = lax.axis_index("c")
    s = lax.axis_index("s")
    base = s * EB
    zb = (zb0, zb1)
    idxg = (idxg0, idxg1)
    idxd = (idxd0, idxd1)
    semg = (semg0, semg1)
    sems = (sems0, sems1)
    seme = (seme0, seme1)
    CQ = SCHK // CH

    pltpu.sync_copy(s1_h, s1v)
    pltpu.sync_copy(s2_h, s2v)

    @pl.loop(0, CH)
    def _zrow(r):
        for q in range(D // 16):
            zb0[r, pl.ds(q * 16, 16)] = jnp.zeros((16,), jnp.float32)

    @pl.loop(0, 1024 // 16)
    def _zden(i):
        dtmp[pl.ds(i * 16, 16)] = jnp.zeros((16,), jnp.float32)

    for k in range(640 // CH):
        pltpu.sync_copy(zb0, aggsh.at[pl.ds(s * 640 + k * CH, CH)])

    @pl.when(c == 0)
    def _():
        pltpu.sync_copy(dtmp.at[pl.ds(0, 640)], dshared.at[pl.ds(s * 640, 640)])

    plsc.subcore_barrier()

    @pl.loop(0, EB // SCHK)
    def _super(u):
        sbase = base + u * SCHK
        pltpu.sync_copy(epk_h.at[s * (EB // SCHK) + u], esv)

        def _stage_gather(p, co):
            @pl.when(c == 0)
            def _():
                for k in range(CH // 16):
                    idxg[p][pl.ds(k * 16, 16)] = esv[0, pl.ds(co + k * 16, 16)]
                pltpu.async_copy(z_h.at[idxg[p]], zb[p], semg[p])

            @pl.when(c == 1)
            def _():
                pltpu.async_copy(ea_h.at[pl.ds(sbase + co, CH)], zb[p],
                                 semg[p])

        def _wait_gather(p):
            pltpu.make_async_copy(ea_h.at[pl.ds(sbase, CH)], zb[p],
                                  semg[p]).wait()

        def _scale(p, co):
            @pl.loop(0, CH // 16)
            def _sc(k):
                o = co + k * 16
                sv = esv[0, pl.ds(o, 16)]
                dv = esv[1, pl.ds(o, 16)]
                s3 = plsc.bitcast(esv[2, pl.ds(o, 16)], jnp.float32)
                g2 = plsc.load_gather(s2v, [dv])
                a = plsc.load_gather(s1v, [sv]) + g2 + s3
                e = jnp.maximum(a, 0.01 * a)
                cc = jnp.maximum(g2, 0.01 * g2)
                exl = jnp.exp(e - cc)
                exv[pl.ds(o, 16)] = exl
                for i in range(16):
                    w = exl[i]
                    e2 = k * 16 + i
                    for q in range(D // 16):
                        zb[p][e2, pl.ds(q * 16, 16)] = (
                            zb[p][e2, pl.ds(q * 16, 16)] * w)

        def _start_scatter(p, co):
            for k in range(CH // 16):
                idxd[p][pl.ds(k * 16, 16)] = esv[1, pl.ds(co + k * 16, 16)]
            pltpu.async_copy(zb[p], aggsh.at[idxd[p]], sems[p], add=True)

            @pl.when(c == 0)
            def _():
                pltpu.async_copy(exv.at[pl.ds(co, CH)], dshared.at[idxd[p]],
                                 seme[p], add=True)

        def _wait_scatter(p):
            pltpu.make_async_copy(zb[p], aggsh.at[idxd[p]], sems[p]).wait()

            @pl.when(c == 0)
            def _():
                pltpu.make_async_copy(exv.at[pl.ds(0, CH)],
                                      dshared.at[idxd[p]], seme[p]).wait()

        _stage_gather(0, 0)

        @pl.loop(0, CQ // 2)
        def _pair(t):
            coa = 2 * t * CH
            cob = coa + CH
            _wait_gather(0)

            @pl.when(t > 0)
            def _():
                _wait_scatter(1)

            _stage_gather(1, cob)
            _scale(0, coa)
            _start_scatter(0, coa)
            _wait_gather(1)

            @pl.when(t < CQ // 2 - 1)
            def _():
                _wait_scatter(0)
                _stage_gather(0, coa + 2 * CH)

            _scale(1, cob)
            _start_scatter(1, cob)

        _wait_scatter(0)
        _wait_scatter(1)

    plsc.subcore_barrier()

    @pl.when(c == 0)
    def _():
        pltpu.sync_copy(aggsh.at[pl.ds(s * 640, 640)],
                        out_h.at[pl.ds(s * 640, 640)])

    @pl.when(c == 1)
    def _():
        pltpu.sync_copy(aggsh.at[pl.ds(s * 640, 640)],
                        out_h.at[pl.ds(NP + s * 640, 640)])

    @pl.when(jnp.logical_and(c == 0, s < 10))
    def _():
        pltpu.sync_copy(dshared.at[pl.ds(s * 1024, 1024)], dtmp)

        @pl.loop(0, 8)
        def _d2(r):
            for q in range(D // 16):
                dtmp2[r, pl.ds(q * 16, 16)] = dtmp[pl.ds(r * 128 + q * 16, 16)]

        pltpu.sync_copy(dtmp2, out_h.at[pl.ds(2 * NP + s * 8, 8)])


def _pack_edges(src, dst, s3row):
    nblk = E // SCHK
    s3b = jax.lax.bitcast_convert_type(s3row, jnp.int32)
    return jnp.stack([jnp.reshape(src, (nblk, SCHK)),
                      jnp.reshape(dst, (nblk, SCHK)),
                      jnp.reshape(s3b, (nblk, SCHK))], axis=1)


def _layer_parts(epk, s1, s2, z, ea):
    agg = _sc_layer(epk, s1, s2, z, ea)
    return agg, jnp.reshape(agg[2 * NP:], (NP, 1))


def kernel(x, edge_index, edge_attr, fc_w0, fc_r_w0, attn_w0, loop_w0,
           fc_w1, fc_r_w1, attn_w1, loop_w1):
    src = edge_index[0]
    dst = edge_index[1]
    xp = jnp.pad(x, ((0, NP - N), (0, 0)))

    s3T = _tc_pre(edge_attr, fc_r_w0, fc_r_w1, attn_w0, attn_w1)
    epk0 = _pack_edges(src, dst, s3T[0])
    epk1 = _pack_edges(src, dst, s3T[1])

    z0, zl0, s12 = _tc0(xp, fc_w0, loop_w0, attn_w0)
    agg0, den0 = _layer_parts(epk0, s12[0], s12[1], z0, edge_attr)

    z1, zl1, s12b = _tc1(agg0, den0, fc_r_w0, zl0, fc_w1, loop_w1, attn_w1)
    agg1, den1 = _layer_parts(epk1, s12b[0], s12b[1], z1, edge_attr)

    return _tc2(agg1, den1, fc_r_w1, zl1)[:N]
